# baseline (speedup 1.0000x reference)
import functools

import jax
import jax.numpy as jnp
from jax import lax
from jax.experimental import pallas as pl
from jax.experimental.pallas import tpu as pltpu
from jax.experimental.pallas import tpu_sc as plsc

N = 32768
F = 6
C = 16
RES = 16
NVOL = 96
NPAIR = 48
NB = 512
P = 1024
NW = 32
JPW = 3
NH = N // 2
ROWP = C
VOXELS = RES * RES * RES * ROWP


def _encode_body(freqs_ref, pts_ref, i0_ref, i1_ref, w_ref, e_ref):
    pts = pts_ref[...]
    strides = (ROWP, ROWP * RES, ROWP * RES * RES)
    for f in range(F):
        fp = pts * freqs_ref[f]
        s = jnp.sin(fp)
        c = jnp.cos(fp)
        for t, v in ((0, s), (1, c)):
            x = (v + 1.0) * (0.5 * (RES - 1))
            i0f = jnp.floor(x)
            w = x - i0f
            r = f * 6 + t * 3
            w_ref[r:r + 3, :] = w
            for a in range(3):
                i0a = i0f[a:a + 1, :].astype(jnp.int32) * strides[a]
                i1a = jnp.minimum(
                    i0f[a:a + 1, :] + 1.0, RES - 1.0
                ).astype(jnp.int32) * strides[a]
                i0_ref[r + a:r + a + 1, :] = i0a
                i1_ref[r + a:r + a + 1, :] = i1a
        e_ref[f * 2:f * 2 + 1, :] = s[0:1, :]
        e_ref[f * 2 + 1:f * 2 + 2, :] = c[0:1, :]


def _encode(pts_t, freqs):
    grid = (N // NB,)
    return pl.pallas_call(
        _encode_body,
        grid=grid,
        in_specs=[
            pl.BlockSpec(memory_space=pltpu.SMEM),
            pl.BlockSpec((3, NB), lambda i: (0, i)),
        ],
        out_specs=[
            pl.BlockSpec((36, NB), lambda i: (0, i)),
            pl.BlockSpec((36, NB), lambda i: (0, i)),
            pl.BlockSpec((36, NB), lambda i: (0, i)),
            pl.BlockSpec((12, NB), lambda i: (0, i)),
        ],
        out_shape=[
            jax.ShapeDtypeStruct((36, N), jnp.int32),
            jax.ShapeDtypeStruct((36, N), jnp.int32),
            jax.ShapeDtypeStruct((36, N), jnp.float32),
            jax.ShapeDtypeStruct((12, N), jnp.float32),
        ],
    )(freqs, pts_t)


@functools.partial(
    pl.kernel,
    mesh=plsc.VectorSubcoreMesh(core_axis_name="c", subcore_axis_name="s"),
    compiler_params=pltpu.CompilerParams(needs_layout_passes=False),
    out_type=jax.ShapeDtypeStruct((NVOL, C, N), jnp.float32),
    scratch_types=[
        pltpu.VMEM((VOXELS,), jnp.int32),
        pltpu.VMEM((P,), jnp.int32),
        pltpu.VMEM((P,), jnp.int32),
        pltpu.VMEM((P,), jnp.int32),
        pltpu.VMEM((P,), jnp.int32),
        pltpu.VMEM((P,), jnp.int32),
        pltpu.VMEM((P,), jnp.int32),
        pltpu.VMEM((P,), jnp.float32),
        pltpu.VMEM((P,), jnp.float32),
        pltpu.VMEM((P,), jnp.float32),
        pltpu.VMEM((P,), jnp.float32),
        pltpu.VMEM((P,), jnp.float32),
        pltpu.VMEM((C, P), jnp.float32),
        pltpu.VMEM((C, P), jnp.float32),
        pltpu.VMEM((C, 16), jnp.int32),
        pltpu.SemaphoreType.DMA,
        pltpu.SemaphoreType.DMA,
    ],
)
def _sample(cv2, i0, i1, w, e, out, vol_v,
            ax0_v, ax1_v, ay0_v, ay1_v, az0_v, az1_v,
            wx_v, wy_v, wz_v, e0_v, e1_v, out0_v, out1_v, dg_v,
            sem_in, sem_out):
    wid = lax.axis_index("s") * 2 + lax.axis_index("c")
    iot = lax.broadcasted_iota(jnp.int32, (16,), 0)
    for j in range(C):
        dg_v[j, :] = (iot + j) & (C - 1)
    for vi in range(JPW):
        job = wid * JPW + vi
        pv = job // 2
        h = job % 2
        f = pv // 8
        co = pv % 8
        tx = co >> 2
        ty = (co >> 1) & 1
        tz = co & 1
        rx = f * 6 + tx * 3
        ry = f * 6 + ty * 3 + 1
        rz = f * 6 + tz * 3 + 2
        re = f * 2
        b0 = f * 16 + co
        nb = h * NH
        pltpu.sync_copy(cv2.at[pv], vol_v)

        def chunk_body(ci, _, rx=rx, ry=ry, rz=rz, re=re, b0=b0, nb=nb):
            n0 = nb + ci * P
            sl = pl.ds(n0, P)
            cps = [
                pltpu.async_copy(i0.at[rx, sl], ax0_v, sem_in),
                pltpu.async_copy(i1.at[rx, sl], ax1_v, sem_in),
                pltpu.async_copy(i0.at[ry, sl], ay0_v, sem_in),
                pltpu.async_copy(i1.at[ry, sl], ay1_v, sem_in),
                pltpu.async_copy(i0.at[rz, sl], az0_v, sem_in),
                pltpu.async_copy(i1.at[rz, sl], az1_v, sem_in),
                pltpu.async_copy(w.at[rx, sl], wx_v, sem_in),
                pltpu.async_copy(w.at[ry, sl], wy_v, sem_in),
                pltpu.async_copy(w.at[rz, sl], wz_v, sem_in),
                pltpu.async_copy(e.at[re, sl], e0_v, sem_in),
                pltpu.async_copy(e.at[re + 1, sl], e1_v, sem_in),
            ]
            for cp in cps:
                cp.wait()

            @plsc.parallel_loop(0, P // 16, 1, unroll=4)
            def grp_body(gi):
                p0 = gi * 16
                gsl = pl.ds(p0, 16)
                ax0 = ax0_v[gsl]
                ax1 = ax1_v[gsl]
                ay0 = ay0_v[gsl]
                ay1 = ay1_v[gsl]
                az0 = az0_v[gsl]
                az1 = az1_v[gsl]
                wx = wx_v[gsl]
                wy = wy_v[gsl]
                wz = wz_v[gsl]
                ev0 = e0_v[gsl]
                ev1 = e1_v[gsl]
                gx0 = 1.0 - wx
                gy0 = 1.0 - wy
                gz0 = 1.0 - wz
                t00 = gz0 * gy0
                t01 = gz0 * wy
                t10 = wz * gy0
                t11 = wz * wy
                w000 = t00 * gx0
                w001 = t00 * wx
                w010 = t01 * gx0
                w011 = t01 * wx
                w100 = t10 * gx0
                w101 = t10 * wx
                w110 = t11 * gx0
                w111 = t11 * wx
                zy00 = az0 + ay0
                zy01 = az0 + ay1
                zy10 = az1 + ay0
                zy11 = az1 + ay1
                k000 = zy00 + ax0
                k001 = zy00 + ax1
                k010 = zy01 + ax0
                k011 = zy01 + ax1
                k100 = zy10 + ax0
                k101 = zy10 + ax1
                k110 = zy11 + ax0
                k111 = zy11 + ax1
                iop0 = iot + p0

                def gat2(kv):
                    word = plsc.load_gather(vol_v, [kv])
                    a = plsc.bitcast(word << 16, jnp.float32)
                    b = plsc.bitcast(word, jnp.float32)
                    return a, b

                kws = ((k000, w000), (k001, w001), (k010, w010),
                       (k011, w011), (k100, w100), (k101, w101),
                       (k110, w110), (k111, w111))
                for j in range(C):
                    dg = dg_v[j, :]
                    r0 = ev0
                    r1 = ev1
                    for kv, wk in kws:
                        av, bv = gat2(kv + dg)
                        r0 = r0 + wk * av
                        r1 = r1 + wk * bv
                    plsc.store_scatter(out0_v, [dg, iop0], r0)
                    plsc.store_scatter(out1_v, [dg, iop0], r1)

            pltpu.async_copy(out0_v, out.at[b0, :, sl], sem_out).wait()
            pltpu.async_copy(out1_v, out.at[b0 + 8, :, sl], sem_out).wait()
            return 0

        lax.fori_loop(0, NH // P, chunk_body, 0)


@jax.jit
def kernel(points, freqs, cv):
    pts_t = points.T
    a = jnp.transpose(cv, (0, 2, 3, 4, 1)).reshape(F, 2, 8, VOXELS)
    u = jax.lax.bitcast_convert_type(
        a.astype(jnp.bfloat16), jnp.uint16).astype(jnp.uint32)
    packed = u[:, 0] | (u[:, 1] << 16)
    cv2 = jax.lax.bitcast_convert_type(
        packed, jnp.int32).reshape(NPAIR, VOXELS)
    i0, i1, w, e = _encode(pts_t, freqs)
    out3 = _sample(cv2, i0, i1, w, e)
    return jnp.transpose(out3, (2, 0, 1)).reshape(N, NVOL * C)


# speedup vs baseline: 1.9018x; 1.9018x over previous
import functools

import jax
import jax.numpy as jnp
from jax import lax
from jax.experimental import pallas as pl
from jax.experimental.pallas import tpu as pltpu
from jax.experimental.pallas import tpu_sc as plsc

N = 32768
F = 6
C = 16
RES = 16
NVOL = 96
NPAIR = 48
NB = 512
P = 1024
NW = 32
JPW = 3
NH = N // 2
ROWP = C
VOXELS = RES * RES * RES * ROWP


def _encode_body(freqs_ref, pts_ref, i0_ref, i1_ref, w_ref, e_ref):
    pts = pts_ref[...]
    strides = (ROWP, ROWP * RES, ROWP * RES * RES)
    for f in range(F):
        fp = pts * freqs_ref[f]
        s = jnp.sin(fp)
        c = jnp.cos(fp)
        for t, v in ((0, s), (1, c)):
            x = (v + 1.0) * (0.5 * (RES - 1))
            i0f = jnp.floor(x)
            w = x - i0f
            r = f * 6 + t * 3
            w_ref[r:r + 3, :] = w
            for a in range(3):
                i0a = i0f[a:a + 1, :].astype(jnp.int32) * strides[a]
                i1a = jnp.minimum(
                    i0f[a:a + 1, :] + 1.0, RES - 1.0
                ).astype(jnp.int32) * strides[a]
                i0_ref[r + a:r + a + 1, :] = i0a
                i1_ref[r + a:r + a + 1, :] = i1a
        e_ref[f * 2:f * 2 + 1, :] = s[0:1, :]
        e_ref[f * 2 + 1:f * 2 + 2, :] = c[0:1, :]


def _encode(pts_t, freqs):
    grid = (N // NB,)
    return pl.pallas_call(
        _encode_body,
        grid=grid,
        in_specs=[
            pl.BlockSpec(memory_space=pltpu.SMEM),
            pl.BlockSpec((3, NB), lambda i: (0, i)),
        ],
        out_specs=[
            pl.BlockSpec((36, NB), lambda i: (0, i)),
            pl.BlockSpec((36, NB), lambda i: (0, i)),
            pl.BlockSpec((36, NB), lambda i: (0, i)),
            pl.BlockSpec((12, NB), lambda i: (0, i)),
        ],
        out_shape=[
            jax.ShapeDtypeStruct((36, N), jnp.int32),
            jax.ShapeDtypeStruct((36, N), jnp.int32),
            jax.ShapeDtypeStruct((36, N), jnp.float32),
            jax.ShapeDtypeStruct((12, N), jnp.float32),
        ],
    )(freqs, pts_t)


@functools.partial(
    pl.kernel,
    mesh=plsc.VectorSubcoreMesh(core_axis_name="c", subcore_axis_name="s"),
    compiler_params=pltpu.CompilerParams(needs_layout_passes=False),
    out_type=jax.ShapeDtypeStruct((NVOL, C, N), jnp.float32),
    scratch_types=[
        pltpu.VMEM((VOXELS,), jnp.int32),
        pltpu.VMEM((P,), jnp.int32),
        pltpu.VMEM((P,), jnp.int32),
        pltpu.VMEM((P,), jnp.int32),
        pltpu.VMEM((P,), jnp.int32),
        pltpu.VMEM((P,), jnp.int32),
        pltpu.VMEM((P,), jnp.int32),
        pltpu.VMEM((P,), jnp.float32),
        pltpu.VMEM((P,), jnp.float32),
        pltpu.VMEM((P,), jnp.float32),
        pltpu.VMEM((P,), jnp.float32),
        pltpu.VMEM((P,), jnp.float32),
        pltpu.VMEM((C, P), jnp.float32),
        pltpu.VMEM((C, P), jnp.float32),
        pltpu.VMEM((C, 16), jnp.int32),
        pltpu.SemaphoreType.DMA,
        pltpu.SemaphoreType.DMA,
    ],
)
def _sample(cv2, i0, i1, w, e, out, vol_v,
            ax0_v, ax1_v, ay0_v, ay1_v, az0_v, az1_v,
            wx_v, wy_v, wz_v, e0_v, e1_v, out0_v, out1_v, dg_v,
            sem_in, sem_out):
    wid = lax.axis_index("s") * 2 + lax.axis_index("c")
    iot = lax.broadcasted_iota(jnp.int32, (16,), 0)
    for j in range(C):
        dg_v[j, :] = (iot + j) & (C - 1)
    for vi in range(JPW):
        job = wid * JPW + vi
        pv = job // 2
        h = job % 2
        f = pv // 8
        co = pv % 8
        tx = co >> 2
        ty = (co >> 1) & 1
        tz = co & 1
        rx = f * 6 + tx * 3
        ry = f * 6 + ty * 3 + 1
        rz = f * 6 + tz * 3 + 2
        re = f * 2
        b0 = f * 16 + co
        nb = h * NH
        pltpu.sync_copy(cv2.at[pv], vol_v)

        def chunk_body(ci, _, rx=rx, ry=ry, rz=rz, re=re, b0=b0, nb=nb):
            n0 = nb + ci * P
            sl = pl.ds(n0, P)
            cps = [
                pltpu.async_copy(i0.at[rx, sl], ax0_v, sem_in),
                pltpu.async_copy(i1.at[rx, sl], ax1_v, sem_in),
                pltpu.async_copy(i0.at[ry, sl], ay0_v, sem_in),
                pltpu.async_copy(i1.at[ry, sl], ay1_v, sem_in),
                pltpu.async_copy(i0.at[rz, sl], az0_v, sem_in),
                pltpu.async_copy(i1.at[rz, sl], az1_v, sem_in),
                pltpu.async_copy(w.at[rx, sl], wx_v, sem_in),
                pltpu.async_copy(w.at[ry, sl], wy_v, sem_in),
                pltpu.async_copy(w.at[rz, sl], wz_v, sem_in),
                pltpu.async_copy(e.at[re, sl], e0_v, sem_in),
                pltpu.async_copy(e.at[re + 1, sl], e1_v, sem_in),
            ]
            for cp in cps:
                cp.wait()

            @plsc.parallel_loop(0, P // 16, 1, unroll=2)
            def grp_body(gi):
                p0 = gi * 16
                gsl = pl.ds(p0, 16)
                ax0 = ax0_v[gsl]
                ax1 = ax1_v[gsl]
                ay0 = ay0_v[gsl]
                ay1 = ay1_v[gsl]
                az0 = az0_v[gsl]
                az1 = az1_v[gsl]
                wx = wx_v[gsl]
                wy = wy_v[gsl]
                wz = wz_v[gsl]
                ev0 = e0_v[gsl]
                ev1 = e1_v[gsl]
                gx0 = 1.0 - wx
                gy0 = 1.0 - wy
                gz0 = 1.0 - wz
                t00 = gz0 * gy0
                t01 = gz0 * wy
                t10 = wz * gy0
                t11 = wz * wy
                w000 = t00 * gx0
                w001 = t00 * wx
                w010 = t01 * gx0
                w011 = t01 * wx
                w100 = t10 * gx0
                w101 = t10 * wx
                w110 = t11 * gx0
                w111 = t11 * wx
                zy00 = az0 + ay0
                zy01 = az0 + ay1
                zy10 = az1 + ay0
                zy11 = az1 + ay1
                k000 = zy00 + ax0
                k001 = zy00 + ax1
                k010 = zy01 + ax0
                k011 = zy01 + ax1
                k100 = zy10 + ax0
                k101 = zy10 + ax1
                k110 = zy11 + ax0
                k111 = zy11 + ax1
                iop0 = iot + p0

                def gat2(kv):
                    word = plsc.load_gather(vol_v, [kv])
                    a = plsc.bitcast(word << 16, jnp.float32)
                    b = plsc.bitcast(word, jnp.float32)
                    return a, b

                kws = ((k000, w000), (k001, w001), (k010, w010),
                       (k011, w011), (k100, w100), (k101, w101),
                       (k110, w110), (k111, w111))
                for j in range(C):
                    dg = dg_v[j, :]
                    r0 = ev0
                    r1 = ev1
                    for kv, wk in kws:
                        av, bv = gat2(kv + dg)
                        r0 = r0 + wk * av
                        r1 = r1 + wk * bv
                    plsc.store_scatter(out0_v, [dg, iop0], r0)
                    plsc.store_scatter(out1_v, [dg, iop0], r1)

            pltpu.async_copy(out0_v, out.at[b0, :, sl], sem_out).wait()
            pltpu.async_copy(out1_v, out.at[b0 + 8, :, sl], sem_out).wait()
            return 0

        lax.fori_loop(0, NH // P, chunk_body, 0)


@jax.jit
def kernel(points, freqs, cv):
    pts_t = points.T
    a = jnp.transpose(cv, (0, 2, 3, 4, 1)).reshape(F, 2, 8, VOXELS)
    u = jax.lax.bitcast_convert_type(
        a.astype(jnp.bfloat16), jnp.uint16).astype(jnp.uint32)
    packed = u[:, 0] | (u[:, 1] << 16)
    cv2 = jax.lax.bitcast_convert_type(
        packed, jnp.int32).reshape(NPAIR, VOXELS)
    i0, i1, w, e = _encode(pts_t, freqs)
    out3 = _sample(cv2, i0, i1, w, e)
    return jnp.transpose(out3, (2, 0, 1)).reshape(N, NVOL * C)
